# baseline (speedup 1.0000x reference)
import functools

import jax
import jax.numpy as jnp
from jax import lax
from jax.experimental import pallas as pl
from jax.experimental.pallas import tpu as pltpu
from jax.experimental.pallas import tpu_sc as plsc

_LANES = 16


def _sc_geometry():
    try:
        info = plsc.get_sparse_core_info()
        return info.num_cores, info.num_subcores
    except Exception:
        return 2, 16


@functools.lru_cache(maxsize=None)
def _make_gather(vocab, emb, batch, num_cores, num_subcores, chunk):
    nw = num_cores * num_subcores
    b_per_w = batch // nw
    n_chunks = b_per_w // chunk
    n_acc = emb // _LANES
    mesh = plsc.VectorSubcoreMesh(core_axis_name="c", subcore_axis_name="s")

    @functools.partial(
        pl.kernel,
        mesh=mesh,
        out_type=[
            jax.ShapeDtypeStruct((batch, emb), jnp.float32),
            jax.ShapeDtypeStruct((nw * _LANES,), jnp.float32),
        ],
        scratch_types=[
            pltpu.VMEM((2 * n_chunks, chunk), jnp.int32),
            pltpu.VMEM((chunk, emb), jnp.float32),
            pltpu.VMEM((chunk, emb), jnp.float32),
            pltpu.VMEM((chunk, emb), jnp.float32),
            pltpu.VMEM((chunk, emb), jnp.float32),
            pltpu.VMEM((_LANES,), jnp.float32),
            pltpu.SemaphoreType.DMA,
            pltpu.SemaphoreType.DMA,
            pltpu.SemaphoreType.DMA,
            pltpu.SemaphoreType.DMA,
        ],
    )
    def gather_kernel(dom_ids, go_ids, tab_d, tab_g, feat_out, part_out,
                      idx, a_d, a_g, b_d, b_g, acc_v,
                      sg_a, sg_b, sw_a, sw_b):
        wid = lax.axis_index("s") * num_cores + lax.axis_index("c")
        base = wid * b_per_w
        for t in range(n_chunks):
            pltpu.sync_copy(dom_ids.at[pl.ds(base + t * chunk, chunk)],
                            idx.at[t])
            pltpu.sync_copy(go_ids.at[pl.ds(base + t * chunk, chunk)],
                            idx.at[n_chunks + t])
        dbufs = [a_d, b_d]
        gbufs = [a_g, b_g]
        gsems = [sg_a, sg_b]
        wsems = [sw_a, sw_b]

        def fire(c):
            p = c % 2
            return (
                pltpu.async_copy(tab_d.at[idx.at[c]], dbufs[p], gsems[p]),
                pltpu.async_copy(tab_g.at[idx.at[n_chunks + c]], gbufs[p],
                                 gsems[p]),
            )

        ghand = [None] * n_chunks
        whand = [None] * n_chunks
        ghand[0] = fire(0)
        accs = tuple(
            jnp.zeros((_LANES,), jnp.float32) for _ in range(n_acc)
        )
        for c in range(n_chunks):
            p = c % 2
            if c + 1 < n_chunks:
                if c - 1 >= 0:
                    whand[c - 1].wait()
                ghand[c + 1] = fire(c + 1)
            ghand[c][0].wait()
            ghand[c][1].wait()
            x_d, x_g = dbufs[p], gbufs[p]

            def body(r2, accs, x_d=x_d, x_g=x_g):
                new = list(accs)
                for u in range(4):
                    r = r2 * 4 + u
                    for k in range(n_acc):
                        dv = x_d[r, pl.ds(k * _LANES, _LANES)]
                        gv = x_g[r, pl.ds(k * _LANES, _LANES)]
                        x_d[r, pl.ds(k * _LANES, _LANES)] = dv * gv
                        new[k] = new[k] + jnp.abs(dv)
                return tuple(new)

            accs = lax.fori_loop(0, chunk // 4, body, accs)
            whand[c] = pltpu.async_copy(
                x_d, feat_out.at[pl.ds(base + c * chunk, chunk)], wsems[p]
            )
        whand[n_chunks - 2].wait()
        whand[n_chunks - 1].wait()
        total = accs[0]
        for k in range(1, n_acc):
            total = total + accs[k]
        acc_v[...] = total
        pltpu.sync_copy(acc_v, part_out.at[pl.ds(wid * _LANES, _LANES)])

    return gather_kernel


def _mlp_body(f_ref, p_ref, w1_ref, b1_ref, w2_ref, b2_ref,
              out_ref, acc_ref, *, inv_count, last):
    i = pl.program_id(0)
    h = jnp.maximum(
        jnp.dot(f_ref[...], w1_ref[...], preferred_element_type=jnp.float32)
        + b1_ref[...],
        0.0,
    )
    out_ref[...] = (
        jnp.dot(h, w2_ref[...], preferred_element_type=jnp.float32)
        + b2_ref[...]
    )

    @pl.when(i == last)
    def _():
        acc_ref[...] = (jnp.sum(p_ref[...]) * inv_count).reshape(1, 1)


@functools.lru_cache(maxsize=None)
def _make_mlp(batch, emb, hidden, n_part, blk):
    grid = batch // blk
    body = functools.partial(
        _mlp_body, inv_count=1.0 / (batch * emb), last=grid - 1
    )
    return pl.pallas_call(
        body,
        grid=(grid,),
        in_specs=[
            pl.BlockSpec((blk, emb), lambda i: (i, 0)),
            pl.BlockSpec((1, n_part), lambda i: (0, 0)),
            pl.BlockSpec((emb, hidden), lambda i: (0, 0)),
            pl.BlockSpec((1, hidden), lambda i: (0, 0)),
            pl.BlockSpec((hidden, 1), lambda i: (0, 0)),
            pl.BlockSpec((1, 1), lambda i: (0, 0)),
        ],
        out_specs=[
            pl.BlockSpec((blk, 1), lambda i: (i, 0)),
            pl.BlockSpec((1, 1), lambda i: (0, 0)),
        ],
        out_shape=[
            jax.ShapeDtypeStruct((batch, 1), jnp.float32),
            jax.ShapeDtypeStruct((1, 1), jnp.float32),
        ],
    )


def kernel(domain_id, go_id, table_domain, table_go, W1, b1, W2, b2):
    batch = domain_id.shape[0]
    vocab, emb = table_domain.shape
    hidden = W1.shape[1]
    num_cores, num_subcores = _sc_geometry()
    nw = num_cores * num_subcores
    chunk = 128
    n_chunks = batch // nw // chunk

    dom_ids = domain_id.astype(jnp.int32)
    go_ids = go_id.astype(jnp.int32)

    gather = _make_gather(vocab, emb, batch, num_cores, num_subcores, chunk)
    feat, partials = gather(dom_ids, go_ids, table_domain, table_go)

    mlp = _make_mlp(batch, emb, hidden, nw * _LANES, 4096)
    out, acc = mlp(
        feat, partials.reshape(1, -1), W1, b1.reshape(1, hidden), W2,
        b2.reshape(1, 1)
    )
    return out, acc.reshape(())


# speedup vs baseline: 1.0418x; 1.0418x over previous
import functools

import jax
import jax.numpy as jnp
from jax import lax
from jax.experimental import pallas as pl
from jax.experimental.pallas import tpu as pltpu
from jax.experimental.pallas import tpu_sc as plsc

_LANES = 16


def _sc_geometry():
    try:
        info = plsc.get_sparse_core_info()
        return info.num_cores, info.num_subcores
    except Exception:
        return 2, 16


@functools.lru_cache(maxsize=None)
def _make_gather(vocab, emb, batch, num_cores, num_subcores, chunk):
    nw = num_cores * num_subcores
    b_per_w = batch // nw
    n_chunks = b_per_w // chunk
    n_acc = emb // _LANES
    mesh = plsc.VectorSubcoreMesh(core_axis_name="c", subcore_axis_name="s")

    @functools.partial(
        pl.kernel,
        mesh=mesh,
        out_type=[
            jax.ShapeDtypeStruct((batch, emb), jnp.float32),
            jax.ShapeDtypeStruct((nw * _LANES,), jnp.float32),
        ],
        scratch_types=[
            pltpu.VMEM((2 * n_chunks, chunk), jnp.int32),
            pltpu.VMEM((chunk, emb), jnp.float32),
            pltpu.VMEM((chunk, emb), jnp.float32),
            pltpu.VMEM((chunk, emb), jnp.float32),
            pltpu.VMEM((chunk, emb), jnp.float32),
            pltpu.VMEM((_LANES,), jnp.float32),
            pltpu.SemaphoreType.DMA,
            pltpu.SemaphoreType.DMA,
            pltpu.SemaphoreType.DMA,
            pltpu.SemaphoreType.DMA,
        ],
    )
    def gather_kernel(dom_ids, go_ids, tab_d, tab_g, feat_out, part_out,
                      idx, a_d, a_g, b_d, b_g, acc_v,
                      sg_a, sg_b, sw_a, sw_b):
        wid = lax.axis_index("s") * num_cores + lax.axis_index("c")
        base = wid * b_per_w
        pltpu.sync_copy(dom_ids.at[wid], idx.at[pl.ds(0, n_chunks)])
        pltpu.sync_copy(go_ids.at[wid], idx.at[pl.ds(n_chunks, n_chunks)])
        dbufs = [a_d, b_d]
        gbufs = [a_g, b_g]
        gsems = [sg_a, sg_b]
        wsems = [sw_a, sw_b]

        def fire(c):
            p = c % 2
            return (
                pltpu.async_copy(tab_d.at[idx.at[c]], dbufs[p], gsems[p]),
                pltpu.async_copy(tab_g.at[idx.at[n_chunks + c]], gbufs[p],
                                 gsems[p]),
            )

        ghand = [None] * n_chunks
        whand = [None] * n_chunks
        ghand[0] = fire(0)
        accs = tuple(
            jnp.zeros((_LANES,), jnp.float32) for _ in range(n_acc)
        )
        for c in range(n_chunks):
            p = c % 2
            if c + 1 < n_chunks:
                if c - 1 >= 0:
                    whand[c - 1].wait()
                ghand[c + 1] = fire(c + 1)
            ghand[c][0].wait()
            ghand[c][1].wait()
            x_d, x_g = dbufs[p], gbufs[p]

            def body(r2, accs, x_d=x_d, x_g=x_g):
                new = list(accs)
                for u in range(4):
                    r = r2 * 4 + u
                    for k in range(n_acc):
                        dv = x_d[r, pl.ds(k * _LANES, _LANES)]
                        gv = x_g[r, pl.ds(k * _LANES, _LANES)]
                        x_d[r, pl.ds(k * _LANES, _LANES)] = dv * gv
                        new[k] = new[k] + jnp.abs(dv)
                return tuple(new)

            accs = lax.fori_loop(0, chunk // 4, body, accs)
            whand[c] = pltpu.async_copy(
                x_d, feat_out.at[pl.ds(base + c * chunk, chunk)], wsems[p]
            )
        whand[n_chunks - 2].wait()
        whand[n_chunks - 1].wait()
        total = accs[0]
        for k in range(1, n_acc):
            total = total + accs[k]
        acc_v[...] = total
        pltpu.sync_copy(acc_v, part_out.at[pl.ds(wid * _LANES, _LANES)])

    return gather_kernel


def _mlp_body(f_ref, p_ref, w1_ref, b1_ref, w2_ref, b2_ref,
              out_ref, acc_ref, *, inv_count, last):
    i = pl.program_id(0)
    h = jnp.maximum(
        jnp.dot(f_ref[...], w1_ref[...], preferred_element_type=jnp.float32)
        + b1_ref[...],
        0.0,
    )
    out_ref[...] = (
        jnp.dot(h, w2_ref[...], preferred_element_type=jnp.float32)
        + b2_ref[...]
    )

    @pl.when(i == last)
    def _():
        acc_ref[...] = (jnp.sum(p_ref[...]) * inv_count).reshape(1, 1)


@functools.lru_cache(maxsize=None)
def _make_mlp(batch, emb, hidden, n_part, blk):
    grid = batch // blk
    body = functools.partial(
        _mlp_body, inv_count=1.0 / (batch * emb), last=grid - 1
    )
    return pl.pallas_call(
        body,
        grid=(grid,),
        in_specs=[
            pl.BlockSpec((blk, emb), lambda i: (i, 0)),
            pl.BlockSpec((1, n_part), lambda i: (0, 0)),
            pl.BlockSpec((emb, hidden), lambda i: (0, 0)),
            pl.BlockSpec((1, hidden), lambda i: (0, 0)),
            pl.BlockSpec((hidden, 1), lambda i: (0, 0)),
            pl.BlockSpec((1, 1), lambda i: (0, 0)),
        ],
        out_specs=[
            pl.BlockSpec((blk, 1), lambda i: (i, 0)),
            pl.BlockSpec((1, 1), lambda i: (0, 0)),
        ],
        out_shape=[
            jax.ShapeDtypeStruct((batch, 1), jnp.float32),
            jax.ShapeDtypeStruct((1, 1), jnp.float32),
        ],
    )


def kernel(domain_id, go_id, table_domain, table_go, W1, b1, W2, b2):
    batch = domain_id.shape[0]
    vocab, emb = table_domain.shape
    hidden = W1.shape[1]
    num_cores, num_subcores = _sc_geometry()
    nw = num_cores * num_subcores
    chunk = 128
    n_chunks = batch // nw // chunk

    dom_ids = domain_id.astype(jnp.int32).reshape(nw, n_chunks, chunk)
    go_ids = go_id.astype(jnp.int32).reshape(nw, n_chunks, chunk)

    gather = _make_gather(vocab, emb, batch, num_cores, num_subcores, chunk)
    feat, partials = gather(dom_ids, go_ids, table_domain, table_go)

    mlp = _make_mlp(batch, emb, hidden, nw * _LANES, 4096)
    out, acc = mlp(
        feat, partials.reshape(1, -1), W1, b1.reshape(1, hidden), W2,
        b2.reshape(1, 1)
    )
    return out, acc.reshape(())


# speedup vs baseline: 1.0670x; 1.0242x over previous
import functools

import jax
import jax.numpy as jnp
from jax import lax
from jax.experimental import pallas as pl
from jax.experimental.pallas import tpu as pltpu
from jax.experimental.pallas import tpu_sc as plsc

_LANES = 16


def _sc_geometry():
    try:
        info = plsc.get_sparse_core_info()
        return info.num_cores, info.num_subcores
    except Exception:
        return 2, 16


@functools.lru_cache(maxsize=None)
def _make_gather(vocab, emb, batch, num_cores, num_subcores, chunk):
    nw = num_cores * num_subcores
    b_per_w = batch // nw
    n_chunks = b_per_w // chunk
    n_acc = emb // _LANES
    mesh = plsc.VectorSubcoreMesh(core_axis_name="c", subcore_axis_name="s")

    @functools.partial(
        pl.kernel,
        mesh=mesh,
        out_type=[
            jax.ShapeDtypeStruct((batch, emb), jnp.float32),
            jax.ShapeDtypeStruct((nw * _LANES,), jnp.float32),
        ],
        scratch_types=[
            pltpu.VMEM((2 * n_chunks, chunk), jnp.int32),
            pltpu.VMEM((chunk, emb), jnp.float32),
            pltpu.VMEM((chunk, emb), jnp.float32),
            pltpu.VMEM((chunk, emb), jnp.float32),
            pltpu.VMEM((chunk, emb), jnp.float32),
            pltpu.VMEM((chunk, emb), jnp.float32),
            pltpu.VMEM((chunk, emb), jnp.float32),
            pltpu.VMEM((_LANES,), jnp.float32),
            pltpu.SemaphoreType.DMA,
            pltpu.SemaphoreType.DMA,
            pltpu.SemaphoreType.DMA,
            pltpu.SemaphoreType.DMA,
            pltpu.SemaphoreType.DMA,
            pltpu.SemaphoreType.DMA,
        ],
    )
    def gather_kernel(dom_ids, go_ids, tab_d, tab_g, feat_out, part_out,
                      idx, a_d, a_g, b_d, b_g, c_d, c_g, acc_v,
                      sg_a, sg_b, sg_c, sw_a, sw_b, sw_c):
        wid = lax.axis_index("s") * num_cores + lax.axis_index("c")
        base = wid * b_per_w
        pltpu.sync_copy(dom_ids.at[wid], idx.at[pl.ds(0, n_chunks)])
        pltpu.sync_copy(go_ids.at[wid], idx.at[pl.ds(n_chunks, n_chunks)])
        dbufs = [a_d, b_d, c_d]
        gbufs = [a_g, b_g, c_g]
        gsems = [sg_a, sg_b, sg_c]
        wsems = [sw_a, sw_b, sw_c]

        def fire(c):
            p = c % 3
            return (
                pltpu.async_copy(tab_d.at[idx.at[c]], dbufs[p], gsems[p]),
                pltpu.async_copy(tab_g.at[idx.at[n_chunks + c]], gbufs[p],
                                 gsems[p]),
            )

        ghand = [None] * n_chunks
        whand = [None] * n_chunks
        ghand[0] = fire(0)
        if n_chunks > 1:
            ghand[1] = fire(1)
        accs = tuple(
            jnp.zeros((_LANES,), jnp.float32) for _ in range(n_acc)
        )
        for c in range(n_chunks):
            p = c % 3
            if c + 2 < n_chunks:
                if c - 1 >= 0:
                    whand[c - 1].wait()
                ghand[c + 2] = fire(c + 2)
            ghand[c][0].wait()
            ghand[c][1].wait()
            x_d, x_g = dbufs[p], gbufs[p]

            def body(r2, accs, x_d=x_d, x_g=x_g):
                new = list(accs)
                for u in range(2):
                    r = r2 * 2 + u
                    for k in range(n_acc):
                        dv = x_d[r, pl.ds(k * _LANES, _LANES)]
                        gv = x_g[r, pl.ds(k * _LANES, _LANES)]
                        x_d[r, pl.ds(k * _LANES, _LANES)] = dv * gv
                        new[k] = new[k] + jnp.abs(dv)
                return tuple(new)

            accs = lax.fori_loop(0, chunk // 2, body, accs)
            whand[c] = pltpu.async_copy(
                x_d, feat_out.at[pl.ds(base + c * chunk, chunk)], wsems[p]
            )
        for c in range(max(0, n_chunks - 3), n_chunks):
            whand[c].wait()
        total = accs[0]
        for k in range(1, n_acc):
            total = total + accs[k]
        acc_v[...] = total
        pltpu.sync_copy(acc_v, part_out.at[pl.ds(wid * _LANES, _LANES)])

    return gather_kernel


def _mlp_body(f_ref, p_ref, w1_ref, b1_ref, w2_ref, b2_ref,
              out_ref, acc_ref, *, inv_count, last):
    i = pl.program_id(0)
    h = jnp.maximum(
        jnp.dot(f_ref[...], w1_ref[...], preferred_element_type=jnp.float32)
        + b1_ref[...],
        0.0,
    )
    out_ref[...] = (
        jnp.dot(h, w2_ref[...], preferred_element_type=jnp.float32)
        + b2_ref[...]
    )

    @pl.when(i == last)
    def _():
        acc_ref[...] = (jnp.sum(p_ref[...]) * inv_count).reshape(1, 1)


@functools.lru_cache(maxsize=None)
def _make_mlp(batch, emb, hidden, n_part, blk):
    grid = batch // blk
    body = functools.partial(
        _mlp_body, inv_count=1.0 / (batch * emb), last=grid - 1
    )
    return pl.pallas_call(
        body,
        grid=(grid,),
        in_specs=[
            pl.BlockSpec((blk, emb), lambda i: (i, 0)),
            pl.BlockSpec((1, n_part), lambda i: (0, 0)),
            pl.BlockSpec((emb, hidden), lambda i: (0, 0)),
            pl.BlockSpec((1, hidden), lambda i: (0, 0)),
            pl.BlockSpec((hidden, 1), lambda i: (0, 0)),
            pl.BlockSpec((1, 1), lambda i: (0, 0)),
        ],
        out_specs=[
            pl.BlockSpec((blk, 1), lambda i: (i, 0)),
            pl.BlockSpec((1, 1), lambda i: (0, 0)),
        ],
        out_shape=[
            jax.ShapeDtypeStruct((batch, 1), jnp.float32),
            jax.ShapeDtypeStruct((1, 1), jnp.float32),
        ],
    )


def kernel(domain_id, go_id, table_domain, table_go, W1, b1, W2, b2):
    batch = domain_id.shape[0]
    vocab, emb = table_domain.shape
    hidden = W1.shape[1]
    num_cores, num_subcores = _sc_geometry()
    nw = num_cores * num_subcores
    chunk = 128
    n_chunks = batch // nw // chunk

    dom_ids = domain_id.astype(jnp.int32).reshape(nw, n_chunks, chunk)
    go_ids = go_id.astype(jnp.int32).reshape(nw, n_chunks, chunk)

    gather = _make_gather(vocab, emb, batch, num_cores, num_subcores, chunk)
    feat, partials = gather(dom_ids, go_ids, table_domain, table_go)

    mlp = _make_mlp(batch, emb, hidden, nw * _LANES, 4096)
    out, acc = mlp(
        feat, partials.reshape(1, -1), W1, b1.reshape(1, hidden), W2,
        b2.reshape(1, 1)
    )
    return out, acc.reshape(())
